# baseline (speedup 1.0000x reference)
import functools

import jax
import jax.numpy as jnp
from jax import lax
from jax.experimental import pallas as pl
from jax.experimental.pallas import tpu as pltpu
from jax.experimental.pallas import tpu_sc as plsc

_HW = 112 * 112
_K = int(0.5 * _HW)
_PLANES = 4 * 192
_NW = 32
_PPW = _PLANES // _NW
_NV = _HW // 16
_PASSES = ((23, 8), (15, 8), (7, 8), (0, 7))


def _sc_body(x_hbm, o_hbm, buf, hist):
    wid = lax.axis_index("s") * 2 + lax.axis_index("c")
    lanes = lax.iota(jnp.int32, 16)
    ones = jnp.ones((16,), jnp.int32)
    zeros16 = jnp.zeros((16,), jnp.int32)

    def per_plane(p_local, _):
        plane = wid * _PPW + p_local
        pltpu.sync_copy(x_hbm.at[plane], buf)

        prefix = jnp.int32(0)
        k_rem = jnp.int32(_K)
        for shift, width in _PASSES:
            def zero_it(j, _):
                hist[pl.ds(j * 16, 16)] = zeros16
                return 0
            lax.fori_loop(0, 256, zero_it, 0)

            def hist_it(i, _):
                v = buf[pl.ds(i * 16, 16)]
                b = (lax.bitcast_convert_type(v, jnp.int32) & 0x7FFFFFFF) >> shift
                sel = (b >> width) == prefix
                fld = b & ((1 << width) - 1)
                plsc.addupdate_scatter(hist, [lanes * 256 + fld], ones,
                                       mask=sel)
                return 0
            lax.fori_loop(0, _NV, hist_it, 0)

            def scan_it(cc, carry):
                suffix, cnt, snext = carry
                c = 15 - cc
                tot = zeros16
                for l in range(16):
                    tot = tot + hist[pl.ds(l * 256 + c * 16, 16)]
                s = lax.rev(plsc.cumsum(lax.rev(tot, (0,))), (0,)) + suffix
                suffix = jnp.max(s)
                cnt = cnt + jnp.sum(jnp.where(s >= k_rem, 1, 0))
                snext = jnp.maximum(snext,
                                    jnp.max(jnp.where(s < k_rem, s, 0)))
                return suffix, cnt, snext

            _, cnt, snext = lax.fori_loop(
                0, 16, scan_it,
                (jnp.int32(0), jnp.int32(0), jnp.int32(0)))
            bstar = cnt - 1
            prefix = (prefix << width) | bstar
            k_rem = k_rem - snext

        thr = prefix

        def mask_it(i, _):
            v = buf[pl.ds(i * 16, 16)]
            ab = lax.bitcast_convert_type(v, jnp.int32) & 0x7FFFFFFF
            buf[pl.ds(i * 16, 16)] = jnp.where(ab >= thr, v, 0.0)
            return 0
        lax.fori_loop(0, _NV, mask_it, 0)
        pltpu.sync_copy(buf, o_hbm.at[plane])
        return 0

    lax.fori_loop(0, _PPW, per_plane, 0)


@functools.partial(jax.jit, static_argnames=())
def _sc_call(x2):
    return pl.kernel(
        _sc_body,
        out_type=jax.ShapeDtypeStruct((_PLANES, _HW), jnp.float32),
        mesh=plsc.VectorSubcoreMesh(core_axis_name="c", subcore_axis_name="s"),
        compiler_params=pltpu.CompilerParams(needs_layout_passes=False),
        scratch_types=[
            pltpu.VMEM((_HW,), jnp.float32),
            pltpu.VMEM((16 * 256,), jnp.int32),
        ],
    )(x2)


def kernel(x):
    B, C, H, W = x.shape
    x2 = x.reshape(B * C, H * W)
    return _sc_call(x2).reshape(B, C, H, W)


# speedup vs baseline: 2.1328x; 2.1328x over previous
import functools

import jax
import jax.numpy as jnp
from jax import lax
from jax.experimental import pallas as pl
from jax.experimental.pallas import tpu as pltpu
from jax.experimental.pallas import tpu_sc as plsc

_HW = 112 * 112
_K = int(0.5 * _HW)
_PLANES = 4 * 192
_NW = 32
_PPW = _PLANES // _NW
_NV = _HW // 16


def _suffix_scan(hist, k_rem):
    def scan_it(cc, carry):
        suffix, cnt, snext = carry
        c = 15 - cc
        t = [hist[pl.ds(l * 256 + c * 16, 16)] for l in range(16)]
        for stride in (8, 4, 2, 1):
            t = [t[j] + t[j + stride] for j in range(stride)]
        s = lax.rev(plsc.cumsum(lax.rev(t[0], (0,))), (0,)) + suffix
        suffix = jnp.max(s)
        cnt = cnt + jnp.sum(jnp.where(s >= k_rem, 1, 0))
        snext = jnp.maximum(snext, jnp.max(jnp.where(s < k_rem, s, 0)))
        return suffix, cnt, snext

    _, cnt, snext = lax.fori_loop(
        0, 16, scan_it, (jnp.int32(0), jnp.int32(0), jnp.int32(0)))
    return cnt - 1, snext


def _zero_hist(hist):
    zeros16 = jnp.zeros((16,), jnp.int32)

    @plsc.parallel_loop(0, 256, unroll=8)
    def zero_it(j):
        hist[pl.ds(j * 16, 16)] = zeros16


def _sc_body(x_hbm, o_hbm, buf, cand, hist):
    wid = lax.axis_index("s") * 2 + lax.axis_index("c")
    lanes256 = lax.iota(jnp.int32, 16) * 256
    ones = jnp.ones((16,), jnp.int32)

    def per_plane(p_local, _):
        plane = wid * _PPW + p_local
        pltpu.sync_copy(x_hbm.at[plane], buf)

        _zero_hist(hist)

        @plsc.parallel_loop(0, _NV, unroll=8)
        def hist1_it(i):
            v = buf[pl.ds(i * 16, 16)]
            b = (lax.bitcast_convert_type(v, jnp.int32) & 0x7FFFFFFF) >> 23
            plsc.addupdate_scatter(hist, [lanes256 + b], ones)

        b1, snext = _suffix_scan(hist, jnp.int32(_K))
        k_rem = jnp.int32(_K) - snext
        prefix = b1

        @plsc.parallel_loop(0, _NV, carry=jnp.int32(0))
        def comp_it(i, off):
            v = buf[pl.ds(i * 16, 16)]
            b = (lax.bitcast_convert_type(v, jnp.int32) & 0x7FFFFFFF) >> 23
            m = b == b1
            plsc.store_compressed(cand.at[pl.ds(off, 16)], v, mask=m)
            return off + jnp.sum(jnp.where(m, 1, 0))

        n_cand = comp_it
        nv2 = (n_cand + 15) >> 4
        valid_base = lax.iota(jnp.int32, 16)

        for shift, width in ((15, 8), (7, 8), (0, 7)):
            _zero_hist(hist)

            def histc_it(i, _, shift=shift, width=width, prefix=prefix):
                v = cand[pl.ds(i * 16, 16)]
                b = (lax.bitcast_convert_type(v, jnp.int32)
                     & 0x7FFFFFFF) >> shift
                sel = ((b >> width) == prefix) & (valid_base + i * 16 < n_cand)
                fld = b & ((1 << width) - 1)
                plsc.addupdate_scatter(hist, [lanes256 + fld], ones, mask=sel)
                return 0

            lax.fori_loop(0, nv2, histc_it, 0)
            bstar, snext = _suffix_scan(hist, k_rem)
            prefix = (prefix << width) | bstar
            k_rem = k_rem - snext

        thr = prefix

        @plsc.parallel_loop(0, _NV, unroll=8)
        def mask_it(i):
            v = buf[pl.ds(i * 16, 16)]
            ab = lax.bitcast_convert_type(v, jnp.int32) & 0x7FFFFFFF
            buf[pl.ds(i * 16, 16)] = jnp.where(ab >= thr, v, 0.0)

        pltpu.sync_copy(buf, o_hbm.at[plane])
        return 0

    lax.fori_loop(0, _PPW, per_plane, 0)


@jax.jit
def _sc_call(x2):
    return pl.kernel(
        _sc_body,
        out_type=jax.ShapeDtypeStruct((_PLANES, _HW), jnp.float32),
        mesh=plsc.VectorSubcoreMesh(core_axis_name="c", subcore_axis_name="s"),
        compiler_params=pltpu.CompilerParams(needs_layout_passes=False),
        scratch_types=[
            pltpu.VMEM((_HW,), jnp.float32),
            pltpu.VMEM((_HW + 16,), jnp.float32),
            pltpu.VMEM((16 * 256,), jnp.int32),
        ],
    )(x2)


def kernel(x):
    B, C, H, W = x.shape
    x2 = x.reshape(B * C, H * W)
    return _sc_call(x2).reshape(B, C, H, W)
